# baseline (speedup 1.0000x reference)
import jax
import jax.numpy as jnp
from jax.experimental import pallas as pl
from jax.experimental.pallas import tpu as pltpu

_NT = 32768
_H = 768
_NE = 64
_BT = 4096
_KS = 2
_HK = _H // _KS


def _body(x_ref, w_ref, rw_ref, se_ref, acc_ref):
    k = pl.program_id(1)
    partial = jax.lax.dot_general(
        w_ref[...], x_ref[...],
        dimension_numbers=(((1,), (1,)), ((), ())),
        preferred_element_type=jnp.float32)

    @pl.when(k == 0)
    def _():
        acc_ref[...] = partial

    @pl.when(k == _KS - 1)
    def _():
        logits = acc_ref[...] + partial
        e_ids = jax.lax.broadcasted_iota(jnp.int32, logits.shape, 0)
        m1 = jnp.max(logits, axis=0, keepdims=True)
        i1 = jnp.min(jnp.where(logits == m1, e_ids, _NE),
                     axis=0, keepdims=True)
        masked = jnp.where(e_ids == i1, -jnp.inf, logits)
        m2 = jnp.max(masked, axis=0, keepdims=True)
        i2 = jnp.min(jnp.where(masked == m2, e_ids, _NE),
                     axis=0, keepdims=True)
        t = jnp.exp(m2 - m1)
        d = 1.0 + t
        rw_ref[...] = jnp.concatenate([1.0 / d, t / d], axis=0)
        se_ref[...] = jnp.concatenate([i1, i2], axis=0)


def kernel(x, W):
    rw_t, se_t = pl.pallas_call(
        _body,
        grid=(_NT // _BT, _KS),
        in_specs=[
            pl.BlockSpec((_BT, _HK), lambda i, k: (i, k)),
            pl.BlockSpec((_NE, _HK), lambda i, k: (0, k)),
        ],
        out_specs=[
            pl.BlockSpec((2, _BT), lambda i, k: (0, i)),
            pl.BlockSpec((2, _BT), lambda i, k: (0, i)),
        ],
        out_shape=[
            jax.ShapeDtypeStruct((2, _NT), jnp.float32),
            jax.ShapeDtypeStruct((2, _NT), jnp.int32),
        ],
        scratch_shapes=[pltpu.VMEM((_NE, _BT), jnp.float32)],
        compiler_params=pltpu.CompilerParams(
            dimension_semantics=("arbitrary", "arbitrary")),
    )(x, W)
    return (rw_t.T, se_t.T)
